# baseline (speedup 1.0000x reference)
import jax
import jax.numpy as jnp
from jax.experimental import pallas as pl

H = 4
C = 128


def _mm_body(x_ref, w_ref, o_ref):
    o_ref[...] = jnp.dot(x_ref[...], w_ref[...],
                         preferred_element_type=jnp.float32)


def _matmul(x, w):
    n, d = x.shape
    _, k = w.shape
    blk = 2000
    return pl.pallas_call(
        _mm_body,
        grid=(n // blk,),
        in_specs=[pl.BlockSpec((blk, d), lambda i: (i, 0)),
                  pl.BlockSpec((d, k), lambda i: (0, 0))],
        out_specs=pl.BlockSpec((blk, k), lambda i: (i, 0)),
        out_shape=jax.ShapeDtypeStruct((n, k), jnp.float32),
    )(x, w)


def _gat(x, src, dst, mask, W, a_s, a_d, b, n):
    xl = _matmul(x, W).reshape(-1, H, C)
    es = (xl * a_s).sum(-1)
    ed = (xl * a_d).sum(-1)
    e = jax.nn.leaky_relu(es[src] + ed[dst], 0.2)
    e = jnp.where(mask[:, None], e, -1e30)
    m = jax.ops.segment_max(e, dst, num_segments=n)
    m = jnp.where(jnp.isfinite(m), m, 0.0)
    ex = jnp.exp(e - m[dst])
    s = jax.ops.segment_sum(ex, dst, num_segments=n)
    alpha = ex / (s[dst] + 1e-16)
    out = jax.ops.segment_sum(xl[src] * alpha[..., None], dst, num_segments=n)
    return out.mean(axis=1) + b


def kernel(node_embeddings, src_index, tgt_index,
           W1, as1, ad1, b1, W2, as2, ad2, b2, W3, as3, ad3, b3):
    B, N, _ = node_embeddings.shape
    n = B * N
    x = node_embeddings.reshape(n, -1)
    off = (jnp.arange(B) * N)[:, None]
    src = (src_index + off).reshape(-1)
    dst = (tgt_index + off).reshape(-1)
    loop = jnp.arange(n)
    mask = jnp.concatenate([src != dst, jnp.ones((n,), dtype=bool)])
    src = jnp.concatenate([src, loop])
    dst = jnp.concatenate([dst, loop])
    h = _gat(x, src, dst, mask, W1, as1, ad1, b1, n)
    h = jax.nn.elu(h)
    h = _gat(h, src, dst, mask, W2, as2, ad2, b2, n)
    h = jax.nn.elu(h)
    h = _gat(h, src, dst, mask, W3, as3, ad3, b3, n)
    return h.reshape(B, N, -1)


# speedup vs baseline: 11.1456x; 11.1456x over previous
import functools

import jax
import jax.numpy as jnp
from jax import lax
from jax.experimental import pallas as pl
from jax.experimental.pallas import tpu as pltpu
from jax.experimental.pallas import tpu_sc as plsc

H = 4
C = 128
HC = H * C
NRAW = 20000
NPAD = 20480
E0 = 320000
E1 = 340000
EPAD = 344064
NC = 2
NS = 16
NW = NC * NS
SH = EPAD // NW
NCH128 = SH // 128
RPT = NPAD // NS
NHALF = NPAD // 2
ACCR = NHALF + 128
RPT2 = NHALF // NS
SH2 = EPAD // NS
NCHB = SH2 // 128

_mesh = plsc.VectorSubcoreMesh(core_axis_name="c", subcore_axis_name="s")
_sc_params = pltpu.CompilerParams(needs_layout_passes=False)


def _proj(x, w_ref, ast_ref, adt_ref, xlt_ref, est_ref, edt_ref):
    xl = jnp.dot(x, w_ref[...], preferred_element_type=jnp.float32)
    est_ref[...] = lax.dot_general(ast_ref[...], xl, (((1,), (1,)), ((), ())),
                                   preferred_element_type=jnp.float32)
    edt_ref[...] = lax.dot_general(adt_ref[...], xl, (((1,), (1,)), ((), ())),
                                   preferred_element_type=jnp.float32)
    for h in range(H):
        xlt_ref[h] = xl[:, h * C:(h + 1) * C]


def _dense_body(x_ref, w_ref, ast_ref, adt_ref, xlt_ref, est_ref, edt_ref):
    _proj(x_ref[...], w_ref, ast_ref, adt_ref, xlt_ref, est_ref, edt_ref)


def _fuse_body(p_ref, b_ref, w_ref, ast_ref, adt_ref, xlt_ref, est_ref,
               edt_ref):
    xx = p_ref[...] * 0.25 + b_ref[...]
    xx = jnp.where(xx > 0, xx, jnp.exp(xx) - 1.0)
    _proj(xx, w_ref, ast_ref, adt_ref, xlt_ref, est_ref, edt_ref)


def _final_body(p_ref, b_ref, o_ref):
    o_ref[...] = p_ref[...] * 0.25 + b_ref[...]


_BLK = 2048
_GRID = NPAD // _BLK

_proj_out = (
    jax.ShapeDtypeStruct((H, NPAD, C), jnp.float32),
    jax.ShapeDtypeStruct((H, NPAD), jnp.float32),
    jax.ShapeDtypeStruct((H, NPAD), jnp.float32),
)
_proj_out_specs = (
    pl.BlockSpec((H, _BLK, C), lambda i: (0, i, 0)),
    pl.BlockSpec((H, _BLK), lambda i: (0, i)),
    pl.BlockSpec((H, _BLK), lambda i: (0, i)),
)
_w_specs = [
    pl.BlockSpec((C, HC), lambda i: (0, 0)),
    pl.BlockSpec((H, HC), lambda i: (0, 0)),
    pl.BlockSpec((H, HC), lambda i: (0, 0)),
]


def _dense(x, w, ast, adt):
    return pl.pallas_call(
        _dense_body,
        grid=(_GRID,),
        in_specs=[pl.BlockSpec((_BLK, C), lambda i: (i, 0))] + _w_specs,
        out_specs=_proj_out_specs,
        out_shape=_proj_out,
    )(x, w, ast, adt)


def _fuse(p, b, w, ast, adt):
    return pl.pallas_call(
        _fuse_body,
        grid=(_GRID,),
        in_specs=[pl.BlockSpec((_BLK, C), lambda i: (i, 0)),
                  pl.BlockSpec((1, C), lambda i: (0, 0))] + _w_specs,
        out_specs=_proj_out_specs,
        out_shape=_proj_out,
    )(p, b, w, ast, adt)


def _final(p, b):
    return pl.pallas_call(
        _final_body,
        grid=(_GRID,),
        in_specs=[pl.BlockSpec((_BLK, C), lambda i: (i, 0)),
                  pl.BlockSpec((1, C), lambda i: (0, 0))],
        out_specs=pl.BlockSpec((_BLK, C), lambda i: (i, 0)),
        out_shape=jax.ShapeDtypeStruct((NPAD, C), jnp.float32),
    )(p, b)


def _sinv_body(s_ref, o_ref):
    o_ref[...] = 1.0 / (s_ref[0][None] + s_ref[1][None] + 1e-30)


def _sinv(s_flat):
    return pl.pallas_call(
        _sinv_body,
        grid=(_GRID,),
        in_specs=[pl.BlockSpec((NC, H * _BLK), lambda i: (0, i))],
        out_specs=pl.BlockSpec((1, H * _BLK), lambda i: (0, i)),
        out_shape=jax.ShapeDtypeStruct((1, H * NPAD), jnp.float32),
    )(s_flat.reshape(NC, H * NPAD))


def _attn_body(src_hbm, dst_hbm, est_hbm, edt_hbm, k_hbm, g_out, s_out,
               src_v, dst_v, es_v, ed_v, s_v, g_v, k_v, tmp_v, s_sh):
    cid = lax.axis_index("c")
    sid = lax.axis_index("s")
    wid = sid * NC + cid
    eb = wid * SH
    pltpu.sync_copy(src_hbm.at[pl.ds(eb, SH)], src_v)
    pltpu.sync_copy(dst_hbm.at[pl.ds(eb, SH)], dst_v)
    pltpu.sync_copy(k_hbm, k_v)
    z16 = jnp.zeros((16,), jnp.float32)
    for h in range(H):
        pltpu.sync_copy(est_hbm.at[pl.ds(h * NPAD, NPAD)], es_v)
        pltpu.sync_copy(edt_hbm.at[pl.ds(h * NPAD, NPAD)], ed_v)

        def _zero(i, _):
            s_v[pl.ds(i * 16, 16)] = z16
            return 0
        lax.fori_loop(0, NPAD // 16, _zero, 0)
        kh = k_v[pl.ds(0, 16)][h]

        def _chunk(j, _):
            base = j * 16
            sv = src_v[pl.ds(base, 16)]
            dv = dst_v[pl.ds(base, 16)]
            ei = eb + base + lax.iota(jnp.int32, 16)
            t = plsc.load_gather(es_v, [sv]) + plsc.load_gather(ed_v, [dv])
            e = jnp.where(t > 0, t, t * jnp.float32(0.2))
            m = ((sv != dv) | (ei >= E0)) & (ei < E1)
            g = jnp.where(m, jnp.exp(e - kh), jnp.float32(0.0))
            g_v[pl.ds(base, 16)] = g
            plsc.addupdate_scatter(s_v, [dv], g)
            return 0
        lax.fori_loop(0, SH // 16, _chunk, 0)
        pltpu.sync_copy(g_v, g_out.at[pl.ds(h * EPAD + eb, SH)])

        pltpu.sync_copy(s_v, s_sh.at[sid])
        plsc.subcore_barrier()
        rb = sid * RPT

        def _qcomb(q, _):
            pltpu.sync_copy(s_sh.at[:, pl.ds(rb + q * 128, 128)], tmp_v)

            def _red(i, _):
                sl = pl.ds(i * 16, 16)
                acc = tmp_v[0, sl]
                for r in range(1, NS):
                    acc = acc + tmp_v[r, sl]
                s_v[pl.ds(q * 128 + i * 16, 16)] = acc
                return 0
            lax.fori_loop(0, 128 // 16, _red, 0)
            return 0
        lax.fori_loop(0, RPT // 128, _qcomb, 0)
        pltpu.sync_copy(s_v.at[pl.ds(0, RPT)],
                        s_out.at[pl.ds((cid * H + h) * NPAD + rb, RPT)])
        plsc.subcore_barrier()


@functools.partial(
    pl.kernel,
    out_type=(jax.ShapeDtypeStruct((H * EPAD,), jnp.float32),
              jax.ShapeDtypeStruct((NC * H * NPAD,), jnp.float32)),
    mesh=_mesh,
    compiler_params=_sc_params,
    scratch_types=[
        pltpu.VMEM((SH,), jnp.int32),
        pltpu.VMEM((SH,), jnp.int32),
        pltpu.VMEM((NPAD,), jnp.float32),
        pltpu.VMEM((NPAD,), jnp.float32),
        pltpu.VMEM((NPAD,), jnp.float32),
        pltpu.VMEM((SH,), jnp.float32),
        pltpu.VMEM((128,), jnp.float32),
        pltpu.VMEM((NS, 128), jnp.float32),
        pltpu.VMEM_SHARED((NS, NPAD), jnp.float32),
    ],
)
def _attn(src_hbm, dst_hbm, est_hbm, edt_hbm, k_hbm, g_out, s_out,
          src_v, dst_v, es_v, ed_v, s_v, g_v, k_v, tmp_v, s_sh):
    _attn_body(src_hbm, dst_hbm, est_hbm, edt_hbm, k_hbm, g_out, s_out,
               src_v, dst_v, es_v, ed_v, s_v, g_v, k_v, tmp_v, s_sh)


def _alpha_body(dst_hbm, g_hbm, sinv_hbm, al_out, dst_v, sinv_v, a_v):
    cid = lax.axis_index("c")
    sid = lax.axis_index("s")
    wid = sid * NC + cid
    eb = wid * SH
    pltpu.sync_copy(dst_hbm.at[pl.ds(eb, SH)], dst_v)
    for h in range(H):
        pltpu.sync_copy(sinv_hbm.at[pl.ds(h * NPAD, NPAD)], sinv_v)
        pltpu.sync_copy(g_hbm.at[pl.ds(h * EPAD + eb, SH)], a_v)

        def _al(j, _):
            sl = pl.ds(j * 16, 16)
            dv = dst_v[sl]
            a_v[sl] = a_v[sl] * plsc.load_gather(sinv_v, [dv])
            return 0
        lax.fori_loop(0, SH // 16, _al, 0)
        pltpu.sync_copy(a_v, al_out.at[pl.ds(h * EPAD + eb, SH)])


@functools.partial(
    pl.kernel,
    out_type=jax.ShapeDtypeStruct((H * EPAD,), jnp.float32),
    mesh=_mesh,
    compiler_params=_sc_params,
    scratch_types=[
        pltpu.VMEM((SH,), jnp.int32),
        pltpu.VMEM((NPAD,), jnp.float32),
        pltpu.VMEM((SH,), jnp.float32),
    ],
)
def _alpha(dst_hbm, g_hbm, sinv_hbm, al_out, dst_v, sinv_v, a_v):
    _alpha_body(dst_hbm, g_hbm, sinv_hbm, al_out, dst_v, sinv_v, a_v)


def _aggr_body(src_hbm, dst_hbm, al_hbm, xl2_hbm, p_out,
               srcr, dstr, idxg, idxs, ar, rb, acc_sh, sem0):
    cid = lax.axis_index("c")
    sid = lax.axis_index("s")
    eb = sid * SH2
    z16 = jnp.zeros((16,), jnp.float32)
    dbase = cid * NHALF

    def _zr(r, _):
        for k in range(8):
            rb[r, pl.ds(k * 16, 16)] = z16
        return 0
    lax.fori_loop(0, 128, _zr, 0)
    zr0 = sid * (ACCR // NS)

    def _za(t, _):
        pltpu.sync_copy(rb, acc_sh.at[pl.ds(zr0 + t * 128, 128)])
        return 0
    lax.fori_loop(0, ACCR // NS // 128, _za, 0)
    pltpu.sync_copy(rb.at[pl.ds(0, ACCR // NS - (ACCR // NS // 128) * 128)],
                    acc_sh.at[pl.ds(zr0 + (ACCR // NS // 128) * 128,
                                    ACCR // NS - (ACCR // NS // 128) * 128)])
    plsc.subcore_barrier()

    for h in range(H):
        def _chunk(j, _):
            pltpu.sync_copy(al_hbm.at[pl.ds(h * EPAD + eb + j * 128, 128)],
                            ar)
            pltpu.sync_copy(src_hbm.at[pl.ds(eb + j * 128, 128)], srcr)
            pltpu.sync_copy(dst_hbm.at[pl.ds(eb + j * 128, 128)], dstr)
            for k in range(8):
                sl = pl.ds(k * 16, 16)
                idxg[sl] = srcr[sl] + h * NPAD
                dl = dstr[sl] - dbase
                oob = (dl < 0) | (dl >= NHALF)
                idxs[sl] = jnp.where(oob, NHALF, dl)
            d0 = pltpu.async_copy(xl2_hbm.at[idxg], rb, sem0)
            d0.wait()

            def _scale(r16, _):
                av = ar[pl.ds(r16 * 16, 16)]
                for rr in range(16):
                    r = r16 * 16 + rr
                    a = av[rr]
                    for k in range(8):
                        sl = pl.ds(k * 16, 16)
                        rb[r, sl] = rb[r, sl] * a
                return 0
            lax.fori_loop(0, 8, _scale, 0)
            pltpu.sync_copy(rb, acc_sh.at[idxs], add=True)
            return 0
        lax.fori_loop(0, NCHB, _chunk, 0)
    plsc.subcore_barrier()
    pltpu.sync_copy(acc_sh.at[pl.ds(sid * RPT2, RPT2)],
                    p_out.at[pl.ds(dbase + sid * RPT2, RPT2)])


@functools.partial(
    pl.kernel,
    out_type=jax.ShapeDtypeStruct((NPAD, C), jnp.float32),
    mesh=_mesh,
    compiler_params=_sc_params,
    scratch_types=[
        pltpu.VMEM((128,), jnp.int32),
        pltpu.VMEM((128,), jnp.int32),
        pltpu.VMEM((128,), jnp.int32),
        pltpu.VMEM((128,), jnp.int32),
        pltpu.VMEM((128,), jnp.float32),
        pltpu.VMEM((128, C), jnp.float32),
        pltpu.VMEM_SHARED((ACCR, C), jnp.float32),
        pltpu.SemaphoreType.DMA,
    ],
)
def _aggr(src_hbm, dst_hbm, al_hbm, xl2_hbm, p_out,
          srcr, dstr, idxg, idxs, ar, rb, acc_sh, sem0):
    _aggr_body(src_hbm, dst_hbm, al_hbm, xl2_hbm, p_out,
               srcr, dstr, idxg, idxs, ar, rb, acc_sh, sem0)


def _blockdiag(a):
    return (a[:, None, :] * jnp.eye(H, dtype=a.dtype)[:, :, None]).reshape(H, HC)


def _layer(xlT, esT, edT, dstp, srcp):
    t = esT.max(axis=1) + edT.max(axis=1)
    k = jnp.where(t > 0, t, 0.2 * t)
    kpad = jnp.pad(k, (0, 124))
    g, s_flat = _attn(srcp, dstp, esT.reshape(-1), edT.reshape(-1), kpad)
    al = _alpha(dstp, g, _sinv(s_flat).reshape(-1))
    p = _aggr(srcp, dstp, al, xlT.reshape(H * NPAD, C))
    return p


def kernel(node_embeddings, src_index, tgt_index,
           W1, as1, ad1, b1, W2, as2, ad2, b2, W3, as3, ad3, b3):
    B, N, D = node_embeddings.shape
    n = B * N
    x = node_embeddings.reshape(n, D)
    xpad = jnp.pad(x, ((0, NPAD - n), (0, 0)))
    off = (jnp.arange(B, dtype=jnp.int32) * N)[:, None]
    src = (src_index.astype(jnp.int32) + off).reshape(-1)
    dst = (tgt_index.astype(jnp.int32) + off).reshape(-1)
    loop = jnp.arange(n, dtype=jnp.int32)
    zpad = jnp.zeros((EPAD - E1,), jnp.int32)
    srcp = jnp.concatenate([src, loop, zpad])
    dstp = jnp.concatenate([dst, loop, zpad])
    xlT, esT, edT = _dense(xpad, W1, _blockdiag(as1), _blockdiag(ad1))
    p = _layer(xlT, esT, edT, dstp, srcp)
    xlT, esT, edT = _fuse(p, b1.reshape(1, C), W2,
                          _blockdiag(as2), _blockdiag(ad2))
    p = _layer(xlT, esT, edT, dstp, srcp)
    xlT, esT, edT = _fuse(p, b2.reshape(1, C), W3,
                          _blockdiag(as3), _blockdiag(ad3))
    p = _layer(xlT, esT, edT, dstp, srcp)
    xf = _final(p, b3.reshape(1, C))
    return xf[:n].reshape(B, N, C)


# speedup vs baseline: 17.6995x; 1.5880x over previous
import functools

import jax
import jax.numpy as jnp
from jax import lax
from jax.experimental import pallas as pl
from jax.experimental.pallas import tpu as pltpu
from jax.experimental.pallas import tpu_sc as plsc

H = 4
C = 128
HC = H * C
NRAW = 20000
NPAD = 20480
E0 = 320000
E1 = 340000
EPAD = 344064
NC = 2
NS = 16
NW = NC * NS
SH = EPAD // NW
NCH128 = SH // 128
RPT = NPAD // NS
NHALF = NPAD // 2
ACCR = NHALF + 128
RPT2 = NHALF // NS
SH2 = EPAD // NS
NCHB = SH2 // 128

_mesh = plsc.VectorSubcoreMesh(core_axis_name="c", subcore_axis_name="s")
_sc_params = pltpu.CompilerParams(needs_layout_passes=False)


def _proj(x, w_ref, ast_ref, adt_ref, xlt_ref, est_ref, edt_ref):
    xl = jnp.dot(x, w_ref[...], preferred_element_type=jnp.float32)
    est_ref[...] = lax.dot_general(ast_ref[...], xl, (((1,), (1,)), ((), ())),
                                   preferred_element_type=jnp.float32)
    edt_ref[...] = lax.dot_general(adt_ref[...], xl, (((1,), (1,)), ((), ())),
                                   preferred_element_type=jnp.float32)
    for h in range(H):
        xlt_ref[h] = xl[:, h * C:(h + 1) * C]


def _dense_body(x_ref, w_ref, ast_ref, adt_ref, xlt_ref, est_ref, edt_ref):
    _proj(x_ref[...], w_ref, ast_ref, adt_ref, xlt_ref, est_ref, edt_ref)


def _fuse_body(p_ref, b_ref, w_ref, ast_ref, adt_ref, xlt_ref, est_ref,
               edt_ref):
    xx = p_ref[...] * 0.25 + b_ref[...]
    xx = jnp.where(xx > 0, xx, jnp.exp(xx) - 1.0)
    _proj(xx, w_ref, ast_ref, adt_ref, xlt_ref, est_ref, edt_ref)


def _final_body(p_ref, b_ref, o_ref):
    o_ref[...] = p_ref[...] * 0.25 + b_ref[...]


_BLK = 2048
_GRID = NPAD // _BLK

_proj_out = (
    jax.ShapeDtypeStruct((H, NPAD, C), jnp.float32),
    jax.ShapeDtypeStruct((H, NPAD), jnp.float32),
    jax.ShapeDtypeStruct((H, NPAD), jnp.float32),
)
_proj_out_specs = (
    pl.BlockSpec((H, _BLK, C), lambda i: (0, i, 0)),
    pl.BlockSpec((H, _BLK), lambda i: (0, i)),
    pl.BlockSpec((H, _BLK), lambda i: (0, i)),
)
_w_specs = [
    pl.BlockSpec((C, HC), lambda i: (0, 0)),
    pl.BlockSpec((H, HC), lambda i: (0, 0)),
    pl.BlockSpec((H, HC), lambda i: (0, 0)),
]


def _dense(x, w, ast, adt):
    return pl.pallas_call(
        _dense_body,
        grid=(_GRID,),
        in_specs=[pl.BlockSpec((_BLK, C), lambda i: (i, 0))] + _w_specs,
        out_specs=_proj_out_specs,
        out_shape=_proj_out,
    )(x, w, ast, adt)


def _fuse(p, b, w, ast, adt):
    return pl.pallas_call(
        _fuse_body,
        grid=(_GRID,),
        in_specs=[pl.BlockSpec((_BLK, C), lambda i: (i, 0)),
                  pl.BlockSpec((1, C), lambda i: (0, 0))] + _w_specs,
        out_specs=_proj_out_specs,
        out_shape=_proj_out,
    )(p, b, w, ast, adt)


def _final(p, b):
    return pl.pallas_call(
        _final_body,
        grid=(_GRID,),
        in_specs=[pl.BlockSpec((_BLK, C), lambda i: (i, 0)),
                  pl.BlockSpec((1, C), lambda i: (0, 0))],
        out_specs=pl.BlockSpec((_BLK, C), lambda i: (i, 0)),
        out_shape=jax.ShapeDtypeStruct((NPAD, C), jnp.float32),
    )(p, b)


def _sinv_body(s_ref, o_ref):
    o_ref[...] = 1.0 / (s_ref[0][None] + s_ref[1][None] + 1e-30)


def _sinv(s_flat):
    return pl.pallas_call(
        _sinv_body,
        grid=(_GRID,),
        in_specs=[pl.BlockSpec((NC, H * _BLK), lambda i: (0, i))],
        out_specs=pl.BlockSpec((1, H * _BLK), lambda i: (0, i)),
        out_shape=jax.ShapeDtypeStruct((1, H * NPAD), jnp.float32),
    )(s_flat.reshape(NC, H * NPAD))


def _attn_body(src_hbm, dst_hbm, est_hbm, edt_hbm, k_hbm, g_out, s_out,
               src_v, dst_v, es_v, ed_v, s_v, g_v, k_v, tmp_v, s_sh):
    cid = lax.axis_index("c")
    sid = lax.axis_index("s")
    wid = sid * NC + cid
    eb = wid * SH
    pltpu.sync_copy(src_hbm.at[pl.ds(eb, SH)], src_v)
    pltpu.sync_copy(dst_hbm.at[pl.ds(eb, SH)], dst_v)
    pltpu.sync_copy(k_hbm, k_v)
    z16 = jnp.zeros((16,), jnp.float32)
    for h in range(H):
        pltpu.sync_copy(est_hbm.at[pl.ds(h * NPAD, NPAD)], es_v)
        pltpu.sync_copy(edt_hbm.at[pl.ds(h * NPAD, NPAD)], ed_v)

        def _zero(i, _):
            s_v[pl.ds(i * 16, 16)] = z16
            return 0
        lax.fori_loop(0, NPAD // 16, _zero, 0)
        kh = k_v[pl.ds(0, 16)][h]

        def _chunk(j, _):
            base = j * 16
            sv = src_v[pl.ds(base, 16)]
            dv = dst_v[pl.ds(base, 16)]
            ei = eb + base + lax.iota(jnp.int32, 16)
            t = plsc.load_gather(es_v, [sv]) + plsc.load_gather(ed_v, [dv])
            e = jnp.where(t > 0, t, t * jnp.float32(0.2))
            m = ((sv != dv) | (ei >= E0)) & (ei < E1)
            g = jnp.where(m, jnp.exp(e - kh), jnp.float32(0.0))
            g_v[pl.ds(base, 16)] = g
            plsc.addupdate_scatter(s_v, [dv], g)
            return 0
        lax.fori_loop(0, SH // 16, _chunk, 0)
        pltpu.sync_copy(g_v, g_out.at[pl.ds(h * EPAD + eb, SH)])

        pltpu.sync_copy(s_v, s_sh.at[sid])
        plsc.subcore_barrier()
        rb = sid * RPT

        def _qcomb(q, _):
            pltpu.sync_copy(s_sh.at[:, pl.ds(rb + q * 128, 128)], tmp_v)

            def _red(i, _):
                sl = pl.ds(i * 16, 16)
                acc = tmp_v[0, sl]
                for r in range(1, NS):
                    acc = acc + tmp_v[r, sl]
                s_v[pl.ds(q * 128 + i * 16, 16)] = acc
                return 0
            lax.fori_loop(0, 128 // 16, _red, 0)
            return 0
        lax.fori_loop(0, RPT // 128, _qcomb, 0)
        pltpu.sync_copy(s_v.at[pl.ds(0, RPT)],
                        s_out.at[pl.ds((cid * H + h) * NPAD + rb, RPT)])
        plsc.subcore_barrier()


@functools.partial(
    pl.kernel,
    out_type=(jax.ShapeDtypeStruct((H * EPAD,), jnp.float32),
              jax.ShapeDtypeStruct((NC * H * NPAD,), jnp.float32)),
    mesh=_mesh,
    compiler_params=_sc_params,
    scratch_types=[
        pltpu.VMEM((SH,), jnp.int32),
        pltpu.VMEM((SH,), jnp.int32),
        pltpu.VMEM((NPAD,), jnp.float32),
        pltpu.VMEM((NPAD,), jnp.float32),
        pltpu.VMEM((NPAD,), jnp.float32),
        pltpu.VMEM((SH,), jnp.float32),
        pltpu.VMEM((128,), jnp.float32),
        pltpu.VMEM((NS, 128), jnp.float32),
        pltpu.VMEM_SHARED((NS, NPAD), jnp.float32),
    ],
)
def _attn(src_hbm, dst_hbm, est_hbm, edt_hbm, k_hbm, g_out, s_out,
          src_v, dst_v, es_v, ed_v, s_v, g_v, k_v, tmp_v, s_sh):
    _attn_body(src_hbm, dst_hbm, est_hbm, edt_hbm, k_hbm, g_out, s_out,
               src_v, dst_v, es_v, ed_v, s_v, g_v, k_v, tmp_v, s_sh)


def _alpha_body(dst_hbm, g_hbm, sinv_hbm, al_out, dst_v, sinv_v, a_v):
    cid = lax.axis_index("c")
    sid = lax.axis_index("s")
    wid = sid * NC + cid
    eb = wid * SH
    pltpu.sync_copy(dst_hbm.at[pl.ds(eb, SH)], dst_v)
    for h in range(H):
        pltpu.sync_copy(sinv_hbm.at[pl.ds(h * NPAD, NPAD)], sinv_v)
        pltpu.sync_copy(g_hbm.at[pl.ds(h * EPAD + eb, SH)], a_v)

        def _al(j, _):
            sl = pl.ds(j * 16, 16)
            dv = dst_v[sl]
            a_v[sl] = a_v[sl] * plsc.load_gather(sinv_v, [dv])
            return 0
        lax.fori_loop(0, SH // 16, _al, 0)
        pltpu.sync_copy(a_v, al_out.at[pl.ds(h * EPAD + eb, SH)])


@functools.partial(
    pl.kernel,
    out_type=jax.ShapeDtypeStruct((H * EPAD,), jnp.float32),
    mesh=_mesh,
    compiler_params=_sc_params,
    scratch_types=[
        pltpu.VMEM((SH,), jnp.int32),
        pltpu.VMEM((NPAD,), jnp.float32),
        pltpu.VMEM((SH,), jnp.float32),
    ],
)
def _alpha(dst_hbm, g_hbm, sinv_hbm, al_out, dst_v, sinv_v, a_v):
    _alpha_body(dst_hbm, g_hbm, sinv_hbm, al_out, dst_v, sinv_v, a_v)


def _aggr_body(srcB_hbm, dstB_hbm, alB_hbm, xl2_hbm, p_out,
               src8, dst8, al8, idxg, idxs, rb0, rb1, acc_sh, sem0, sem1):
    cid = lax.axis_index("c")
    sid = lax.axis_index("s")
    z16 = jnp.zeros((16,), jnp.float32)
    dbase = cid * NHALF

    def _zr(r, _):
        for k in range(8):
            rb0[r, pl.ds(k * 16, 16)] = z16
        return 0
    lax.fori_loop(0, 128, _zr, 0)
    zr0 = sid * (ACCR // NS)

    def _za(t, _):
        pltpu.sync_copy(rb0, acc_sh.at[pl.ds(zr0 + t * 128, 128)])
        return 0
    lax.fori_loop(0, ACCR // NS // 128, _za, 0)
    pltpu.sync_copy(rb0.at[pl.ds(0, ACCR // NS - (ACCR // NS // 128) * 128)],
                    acc_sh.at[pl.ds(zr0 + (ACCR // NS // 128) * 128,
                                    ACCR // NS - (ACCR // NS // 128) * 128)])
    plsc.subcore_barrier()

    NBLK = NCHB // 8

    def _gidx(h, jp, pp):
        for k in range(8):
            sl = pl.ds(k * 16, 16)
            idxg[pp, sl] = src8[jp, sl] + h * NPAD

    def _scale_scatter(rb, jp):
        def _scale(r16, _):
            av = al8[jp, pl.ds(r16 * 16, 16)]
            for rr in range(16):
                r = r16 * 16 + rr
                a = av[rr]
                for k in range(8):
                    sl = pl.ds(k * 16, 16)
                    rb[r, sl] = rb[r, sl] * a
            return 0
        lax.fori_loop(0, 8, _scale, 0)
        pltpu.sync_copy(rb, acc_sh.at[idxs.at[jp]], add=True)

    def _blk(bj, _):
        blkid = sid * NBLK + bj
        pltpu.sync_copy(srcB_hbm.at[blkid], src8)
        pltpu.sync_copy(dstB_hbm.at[blkid], dst8)

        def _sidx(jj, _):
            for k in range(8):
                sl = pl.ds(k * 16, 16)
                dl = dst8[jj, sl] - dbase
                oob = (dl < 0) | (dl >= NHALF)
                idxs[jj, sl] = jnp.where(oob, NHALF, dl)
            return 0
        lax.fori_loop(0, 8, _sidx, 0)
        for h in range(H):
            pltpu.sync_copy(alB_hbm.at[h].at[blkid], al8)
            _gidx(h, 0, 0)
            pltpu.async_copy(xl2_hbm.at[idxg.at[0]], rb0, sem0)

            def _pair(p, _):
                _gidx(h, 2 * p + 1, 1)
                pltpu.async_copy(xl2_hbm.at[idxg.at[1]], rb1, sem1)
                pltpu.make_async_copy(xl2_hbm.at[idxg.at[0]], rb0,
                                      sem0).wait()
                _scale_scatter(rb0, 2 * p)

                @pl.when(p < 3)
                def _():
                    _gidx(h, 2 * p + 2, 0)
                    pltpu.async_copy(xl2_hbm.at[idxg.at[0]], rb0, sem0)
                pltpu.make_async_copy(xl2_hbm.at[idxg.at[1]], rb1,
                                      sem1).wait()
                _scale_scatter(rb1, 2 * p + 1)
                return 0
            lax.fori_loop(0, 4, _pair, 0)
        return 0
    lax.fori_loop(0, NBLK, _blk, 0)
    plsc.subcore_barrier()
    pltpu.sync_copy(acc_sh.at[pl.ds(sid * RPT2, RPT2)],
                    p_out.at[pl.ds(dbase + sid * RPT2, RPT2)])


@functools.partial(
    pl.kernel,
    out_type=jax.ShapeDtypeStruct((NPAD, C), jnp.float32),
    mesh=_mesh,
    compiler_params=_sc_params,
    scratch_types=[
        pltpu.VMEM((8, 128), jnp.int32),
        pltpu.VMEM((8, 128), jnp.int32),
        pltpu.VMEM((8, 128), jnp.float32),
        pltpu.VMEM((2, 128), jnp.int32),
        pltpu.VMEM((8, 128), jnp.int32),
        pltpu.VMEM((128, C), jnp.float32),
        pltpu.VMEM((128, C), jnp.float32),
        pltpu.VMEM_SHARED((ACCR, C), jnp.float32),
        pltpu.SemaphoreType.DMA,
        pltpu.SemaphoreType.DMA,
    ],
)
def _aggr(srcB_hbm, dstB_hbm, alB_hbm, xl2_hbm, p_out,
          src8, dst8, al8, idxg, idxs, rb0, rb1, acc_sh, sem0, sem1):
    _aggr_body(srcB_hbm, dstB_hbm, alB_hbm, xl2_hbm, p_out,
               src8, dst8, al8, idxg, idxs, rb0, rb1, acc_sh, sem0, sem1)


def _blockdiag(a):
    return (a[:, None, :] * jnp.eye(H, dtype=a.dtype)[:, :, None]).reshape(H, HC)


def _layer(xlT, esT, edT, dstp, srcp):
    t = esT.max(axis=1) + edT.max(axis=1)
    k = jnp.where(t > 0, t, 0.2 * t)
    kpad = jnp.pad(k, (0, 124))
    g, s_flat = _attn(srcp, dstp, esT.reshape(-1), edT.reshape(-1), kpad)
    al = _alpha(dstp, g, _sinv(s_flat).reshape(-1))
    p = _aggr(srcp.reshape(EPAD // 1024, 8, 128),
              dstp.reshape(EPAD // 1024, 8, 128),
              al.reshape(H, EPAD // 1024, 8, 128),
              xlT.reshape(H * NPAD, C))
    return p


def kernel(node_embeddings, src_index, tgt_index,
           W1, as1, ad1, b1, W2, as2, ad2, b2, W3, as3, ad3, b3):
    B, N, D = node_embeddings.shape
    n = B * N
    x = node_embeddings.reshape(n, D)
    xpad = jnp.pad(x, ((0, NPAD - n), (0, 0)))
    off = (jnp.arange(B, dtype=jnp.int32) * N)[:, None]
    src = (src_index.astype(jnp.int32) + off).reshape(-1)
    dst = (tgt_index.astype(jnp.int32) + off).reshape(-1)
    loop = jnp.arange(n, dtype=jnp.int32)
    zpad = jnp.zeros((EPAD - E1,), jnp.int32)
    srcp = jnp.concatenate([src, loop, zpad])
    dstp = jnp.concatenate([dst, loop, zpad])
    xlT, esT, edT = _dense(xpad, W1, _blockdiag(as1), _blockdiag(ad1))
    p = _layer(xlT, esT, edT, dstp, srcp)
    xlT, esT, edT = _fuse(p, b1.reshape(1, C), W2,
                          _blockdiag(as2), _blockdiag(ad2))
    p = _layer(xlT, esT, edT, dstp, srcp)
    xlT, esT, edT = _fuse(p, b2.reshape(1, C), W3,
                          _blockdiag(as3), _blockdiag(ad3))
    p = _layer(xlT, esT, edT, dstp, srcp)
    xf = _final(p, b3.reshape(1, C))
    return xf[:n].reshape(B, N, C)
